# baseline (speedup 1.0000x reference)
import functools

import jax
import jax.numpy as jnp
from jax import lax
from jax.experimental import pallas as pl
from jax.experimental.pallas import tpu as pltpu
from jax.experimental.pallas import tpu_sc as plsc

B = 16384
D = 16
NF = 5
NC, NS = 2, 16
NW = NC * NS
BPW = B // NW
CHUNK = 128
NCH = BPW // CHUNK

FIELDS = ((0, 0), (1, 3), (2, 4), (3, 1), (3, 2))

_MESH = plsc.VectorSubcoreMesh(
    core_axis_name="c", subcore_axis_name="s", num_cores=NC, num_subcores=NS
)


@functools.partial(
    pl.kernel,
    out_type=jax.ShapeDtypeStruct((B, NF * D), jnp.float32),
    mesh=_MESH,
    scratch_types=[
        pltpu.VMEM((NF, NCH, CHUNK), jnp.int32),
        pltpu.VMEM((NF, NCH, CHUNK, D), jnp.float32),
        pltpu.VMEM((BPW, NF * D), jnp.float32),
        pltpu.SemaphoreType.DMA,
        pltpu.SemaphoreType.DMA,
    ],
    compiler_params=pltpu.CompilerParams(
        use_tc_tiling_on_sc=False, needs_layout_passes=False
    ),
)
def _seg_embed(idsT_hbm, w_cls, w_len, w_rad, w_loc, out_hbm,
               idx_v, g_v, out_v, isem, gsem):
    tables = (w_cls, w_len, w_rad, w_loc)
    wid = lax.axis_index("c") * NS + lax.axis_index("s")
    base = pl.multiple_of(wid * BPW, BPW)

    icopies = []
    for f in range(NF):
        for j in range(NCH):
            c = pltpu.make_async_copy(
                idsT_hbm.at[pl.ds(f * B + base + j * CHUNK, CHUNK)],
                idx_v.at[f, j],
                isem,
            )
            c.start()
            icopies.append(c)
    for c in icopies:
        c.wait()

    gcopies = []
    for f, (t, _) in enumerate(FIELDS):
        tab = tables[t]
        for j in range(NCH):
            c = pltpu.make_async_copy(
                tab.at[idx_v.at[f, j]],
                g_v.at[f, j],
                gsem,
            )
            c.start()
            gcopies.append(c)
    for c in gcopies:
        c.wait()

    for f in range(NF):
        for j in range(NCH):
            for r in range(CHUNK):
                v = g_v[f, j, r]
                out_v[j * CHUNK + r, pl.ds(f * D, D)] = v

    pltpu.sync_copy(out_v, out_hbm.at[pl.ds(base, BPW)])


def kernel(inputs, W_cls, W_length, W_radian, W_loc):
    idsT = inputs.astype(jnp.int32).T
    idsT = idsT[jnp.array([c for (_, c) in FIELDS])].reshape(-1)
    return _seg_embed(idsT, W_cls, W_length, W_radian, W_loc)


# speedup vs baseline: 1.1939x; 1.1939x over previous
import functools

import jax
import jax.numpy as jnp
from jax import lax
from jax.experimental import pallas as pl
from jax.experimental.pallas import tpu as pltpu
from jax.experimental.pallas import tpu_sc as plsc

B = 16384
D = 16
NF = 5
NC, NS = 2, 16
NW = NC * NS
BPW = B // NW
CHUNK = 128
NCH = BPW // CHUNK

FIELDS = ((0, 0), (1, 3), (2, 4), (3, 1), (3, 2))

_MESH = plsc.VectorSubcoreMesh(
    core_axis_name="c", subcore_axis_name="s", num_cores=NC, num_subcores=NS
)


@functools.partial(
    pl.kernel,
    out_type=jax.ShapeDtypeStruct((B, NF * D), jnp.float32),
    mesh=_MESH,
    scratch_types=[
        pltpu.VMEM((NF, NCH, CHUNK), jnp.int32),
        pltpu.VMEM((NF, NCH, CHUNK, D), jnp.float32),
        pltpu.VMEM((BPW, NF * D), jnp.float32),
        pltpu.SemaphoreType.DMA,
        pltpu.SemaphoreType.DMA,
    ],
    compiler_params=pltpu.CompilerParams(
        use_tc_tiling_on_sc=False, needs_layout_passes=False
    ),
)
def _seg_embed(idsT_hbm, w_cls, w_len, w_rad, w_loc, out_hbm,
               idx_v, g_v, out_v, isem, gsem):
    tables = (w_cls, w_len, w_rad, w_loc)
    wid = lax.axis_index("c") * NS + lax.axis_index("s")
    base = pl.multiple_of(wid * BPW, BPW)

    icopies = []
    for f in range(NF):
        for j in range(NCH):
            c = pltpu.make_async_copy(
                idsT_hbm.at[pl.ds(f * B + base + j * CHUNK, CHUNK)],
                idx_v.at[f, j],
                isem,
            )
            c.start()
            icopies.append(c)
    for c in icopies:
        c.wait()

    gcopies = []
    for f, (t, _) in enumerate(FIELDS):
        tab = tables[t]
        for j in range(NCH):
            c = pltpu.make_async_copy(
                tab.at[idx_v.at[f, j]],
                g_v.at[f, j],
                gsem,
            )
            c.start()
            gcopies.append(c)
    for c in gcopies:
        c.wait()

    for f in range(NF):
        for j in range(NCH):
            for r in range(CHUNK):
                v = g_v[f, j, r]
                out_v[j * CHUNK + r, pl.ds(f * D, D)] = v

    pltpu.sync_copy(out_v, out_hbm.at[pl.ds(base, BPW)])


_TCOLS = 8192


def _transpose_block(wt_ref, out_ref):
    y = wt_ref[...].T.reshape(_TCOLS // 8, 8, D)
    out_ref[...] = jnp.concatenate([y[:, j, :] for j in range(8)], axis=-1)


_NROWS = 1000000


def _row_major(W):
    wt = W.T
    out2 = pl.pallas_call(
        _transpose_block,
        grid=((_NROWS + _TCOLS - 1) // _TCOLS,),
        in_specs=[pl.BlockSpec((D, _TCOLS), lambda i: (0, i))],
        out_specs=pl.BlockSpec((_TCOLS * D // 128, 128), lambda i: (i, 0)),
        out_shape=jax.ShapeDtypeStruct((_NROWS * D // 128, 128), jnp.float32),
    )(wt)
    return out2.reshape(_NROWS, D)


def kernel(inputs, W_cls, W_length, W_radian, W_loc):
    idsT = inputs.astype(jnp.int32).T
    idsT = idsT[jnp.array([c for (_, c) in FIELDS])].reshape(-1)
    return _seg_embed(idsT, _row_major(W_cls), _row_major(W_length),
                      _row_major(W_radian), _row_major(W_loc))


# speedup vs baseline: 3.2130x; 2.6912x over previous
import functools

import jax
import jax.numpy as jnp
from jax import lax
from jax.experimental import pallas as pl
from jax.experimental.pallas import tpu as pltpu
from jax.experimental.pallas import tpu_sc as plsc

B = 16384
D = 16
NF = 5
NC, NS = 2, 16
NW = NC * NS
BPW = B // NW
CHUNK = 128
NCH = BPW // CHUNK

FIELDS = ((0, 0), (1, 3), (2, 4), (3, 1), (3, 2))

_MESH = plsc.VectorSubcoreMesh(
    core_axis_name="c", subcore_axis_name="s", num_cores=NC, num_subcores=NS
)


@functools.partial(
    pl.kernel,
    out_type=jax.ShapeDtypeStruct((B, NF * D), jnp.float32),
    mesh=_MESH,
    scratch_types=[
        pltpu.VMEM((NF, NCH, CHUNK), jnp.int32),
        pltpu.VMEM((NF, NCH, CHUNK, D), jnp.float32),
        pltpu.VMEM((BPW, NF * D), jnp.float32),
        pltpu.SemaphoreType.DMA,
        pltpu.SemaphoreType.DMA,
    ],
    compiler_params=pltpu.CompilerParams(
        use_tc_tiling_on_sc=False, needs_layout_passes=False
    ),
)
def _seg_embed(idsT_hbm, w_cls, w_len, w_rad, w_loc, out_hbm,
               idx_v, g_v, out_v, isem, gsem):
    tables = (w_cls, w_len, w_rad, w_loc)
    wid = lax.axis_index("c") * NS + lax.axis_index("s")
    base = pl.multiple_of(wid * BPW, BPW)

    icopies = []
    for f in range(NF):
        for j in range(NCH):
            c = pltpu.make_async_copy(
                idsT_hbm.at[pl.ds(f * B + base + j * CHUNK, CHUNK)],
                idx_v.at[f, j],
                isem,
            )
            c.start()
            icopies.append(c)
    for c in icopies:
        c.wait()

    gcopies = []
    for f, (t, _) in enumerate(FIELDS):
        tab = tables[t]
        for j in range(NCH):
            c = pltpu.make_async_copy(
                tab.at[idx_v.at[f, j]],
                g_v.at[f, j],
                gsem,
            )
            c.start()
            gcopies.append(c)
    for c in gcopies:
        c.wait()

    for f in range(NF):
        for j in range(NCH):
            for r in range(CHUNK):
                v = g_v[f, j, r]
                out_v[j * CHUNK + r, pl.ds(f * D, D)] = v

    pltpu.sync_copy(out_v, out_hbm.at[pl.ds(base, BPW)])


_TCOLS = 8192
_NROWS = 1000000
_NBLK = (_NROWS + _TCOLS - 1) // _TCOLS
_PROWS = _NBLK * _TCOLS


def _transpose_block(wt_ref, out_ref):
    x3 = wt_ref[...].reshape(D, _TCOLS // 128, 128)
    for pp in range(_TCOLS // 1024):
        z = jnp.concatenate(
            [x3[:, pp * 8 + q, :] for q in range(8)], axis=0)
        out_ref[pl.ds(pp * 128, 128), :] = z.T


def _row_major(W):
    wt = W.T
    out2 = pl.pallas_call(
        _transpose_block,
        grid=(_NBLK,),
        in_specs=[pl.BlockSpec((D, _TCOLS), lambda i: (0, i))],
        out_specs=pl.BlockSpec((_TCOLS * D // 128, 128), lambda i: (i, 0)),
        out_shape=jax.ShapeDtypeStruct((_PROWS * D // 128, 128), jnp.float32),
    )(wt)
    return out2.reshape(_PROWS, D)


def kernel(inputs, W_cls, W_length, W_radian, W_loc):
    idsT = inputs.astype(jnp.int32).T
    idsT = idsT[jnp.array([c for (_, c) in FIELDS])].reshape(-1)
    idsT = ((idsT & ~1023) | ((idsT & 127) << 3) | ((idsT >> 7) & 7))
    return _seg_embed(idsT, _row_major(W_cls), _row_major(W_length),
                      _row_major(W_radian), _row_major(W_loc))


# speedup vs baseline: 6.4820x; 2.0174x over previous
import functools

import jax
import jax.numpy as jnp
from jax import lax
from jax.experimental import pallas as pl
from jax.experimental.pallas import tpu as pltpu
from jax.experimental.pallas import tpu_sc as plsc

B = 16384
D = 16
NF = 5
NC, NS = 2, 16
NW = NC * NS
BPW = B // NW
CHUNK = 128
NCH = BPW // CHUNK

FIELDS = ((0, 0), (1, 3), (2, 4), (3, 1), (3, 2))

_MESH = plsc.VectorSubcoreMesh(
    core_axis_name="c", subcore_axis_name="s", num_cores=NC, num_subcores=NS
)


@functools.partial(
    pl.kernel,
    out_type=jax.ShapeDtypeStruct((B, NF * D), jnp.float32),
    mesh=_MESH,
    scratch_types=[
        pltpu.VMEM((NF, NCH, CHUNK), jnp.int32),
        pltpu.VMEM((NF, NCH, CHUNK, D), jnp.float32),
        pltpu.VMEM((BPW, NF * D), jnp.float32),
        pltpu.SemaphoreType.DMA,
        pltpu.SemaphoreType.DMA,
    ],
    compiler_params=pltpu.CompilerParams(
        use_tc_tiling_on_sc=False, needs_layout_passes=False
    ),
)
def _seg_embed(idsT_hbm, w_cls, w_len, w_rad, w_loc, out_hbm,
               idx_v, g_v, out_v, isem, gsem):
    tables = (w_cls, w_len, w_rad, w_loc)
    wid = lax.axis_index("c") * NS + lax.axis_index("s")
    base = pl.multiple_of(wid * BPW, BPW)

    icopies = []
    for f in range(NF):
        for j in range(NCH):
            c = pltpu.make_async_copy(
                idsT_hbm.at[pl.ds(f * B + base + j * CHUNK, CHUNK)],
                idx_v.at[f, j],
                isem,
            )
            c.start()
            icopies.append(c)
    for c in icopies:
        c.wait()

    gcopies = []
    for f, (t, _) in enumerate(FIELDS):
        tab = tables[t]
        for j in range(NCH):
            c = pltpu.make_async_copy(
                tab.at[idx_v.at[f, j]],
                g_v.at[f, j],
                gsem,
            )
            c.start()
            gcopies.append(c)
    for c in gcopies:
        c.wait()

    for f in range(NF):
        for j in range(NCH):
            for r in range(CHUNK):
                v = g_v[f, j, r]
                out_v[j * CHUNK + r, pl.ds(f * D, D)] = v

    pltpu.sync_copy(out_v, out_hbm.at[pl.ds(base, BPW)])


_TCOLS = 16384
_NROWS = 1000000
_NBLK = (_NROWS + _TCOLS - 1) // _TCOLS
_PROWS = _NBLK * _TCOLS


def _transpose_block(w0, w1, w2, w3, o0, o1, o2, o3):
    for wt_ref, out_ref in ((w0, o0), (w1, o1), (w2, o2), (w3, o3)):
        x3 = wt_ref[...].reshape(D, _TCOLS // 128, 128)
        for pp in range(_TCOLS // 1024):
            z = jnp.concatenate(
                [x3[:, pp * 8 + q, :] for q in range(8)], axis=0)
            out_ref[pl.ds(pp * 128, 128), :] = z.T


def _row_major4(W_cls, W_length, W_radian, W_loc):
    ospec = jax.ShapeDtypeStruct((_PROWS * D // 128, 128), jnp.float32)
    outs = pl.pallas_call(
        _transpose_block,
        grid=(_NBLK,),
        in_specs=[pl.BlockSpec((D, _TCOLS), lambda i: (0, i))] * 4,
        out_specs=[pl.BlockSpec((_TCOLS * D // 128, 128), lambda i: (i, 0))] * 4,
        out_shape=[ospec] * 4,
    )(W_cls.T, W_length.T, W_radian.T, W_loc.T)
    return [o.reshape(_PROWS, D) for o in outs]


def kernel(inputs, W_cls, W_length, W_radian, W_loc):
    idsT = inputs.astype(jnp.int32).T
    idsT = idsT[jnp.array([c for (_, c) in FIELDS])].reshape(-1)
    idsT = ((idsT & ~1023) | ((idsT & 127) << 3) | ((idsT >> 7) & 7))
    return _seg_embed(idsT, *_row_major4(W_cls, W_length, W_radian, W_loc))


# speedup vs baseline: 7.0362x; 1.0855x over previous
import functools

import jax
import jax.numpy as jnp
from jax import lax
from jax.experimental import pallas as pl
from jax.experimental.pallas import tpu as pltpu
from jax.experimental.pallas import tpu_sc as plsc

B = 16384
D = 16
NF = 5
NC, NS = 2, 16
NW = NC * NS
BPW = B // NW
CHUNK = 128
NCH = BPW // CHUNK

FIELDS = ((0, 0), (1, 3), (2, 4), (3, 1), (3, 2))

_MESH = plsc.VectorSubcoreMesh(
    core_axis_name="c", subcore_axis_name="s", num_cores=NC, num_subcores=NS
)


@functools.partial(
    pl.kernel,
    out_type=jax.ShapeDtypeStruct((B, NF * D), jnp.float32),
    mesh=_MESH,
    scratch_types=[
        pltpu.VMEM((NF, NCH, CHUNK), jnp.int32),
        pltpu.VMEM((NF, NCH, CHUNK, D), jnp.float32),
        pltpu.VMEM((BPW, NF * D), jnp.float32),
        pltpu.SemaphoreType.DMA,
        pltpu.SemaphoreType.DMA,
    ],
    compiler_params=pltpu.CompilerParams(
        use_tc_tiling_on_sc=False, needs_layout_passes=False
    ),
)
def _seg_embed(idsT_hbm, w_cls, w_len, w_rad, w_loc, out_hbm,
               idx_v, g_v, out_v, isem, gsem):
    tables = (w_cls, w_len, w_rad, w_loc)
    wid = lax.axis_index("c") * NS + lax.axis_index("s")
    base = pl.multiple_of(wid * BPW, BPW)

    icopies = []
    for f in range(NF):
        for j in range(NCH):
            c = pltpu.make_async_copy(
                idsT_hbm.at[pl.ds(f * B + base + j * CHUNK, CHUNK)],
                idx_v.at[f, j],
                isem,
            )
            c.start()
            icopies.append(c)
    for c in icopies:
        c.wait()

    gcopies = []
    for f, (t, _) in enumerate(FIELDS):
        tab = tables[t]
        for j in range(NCH):
            c = pltpu.make_async_copy(
                tab.at[idx_v.at[f, j]],
                g_v.at[f, j],
                gsem,
            )
            c.start()
            gcopies.append(c)
    for c in gcopies:
        c.wait()

    for f in range(NF):
        for j in range(NCH):
            for r in range(CHUNK):
                v = g_v[f, j, r]
                out_v[j * CHUNK + r, pl.ds(f * D, D)] = v

    pltpu.sync_copy(out_v, out_hbm.at[pl.ds(base, BPW)])


_TCOLS = 32768
_NROWS = 1000000
_NBLK = (_NROWS + _TCOLS - 1) // _TCOLS
_PROWS = _NBLK * _TCOLS


def _transpose_block(w0, w1, w2, w3, o0, o1, o2, o3):
    for wt_ref, out_ref in ((w0, o0), (w1, o1), (w2, o2), (w3, o3)):
        x3 = wt_ref[...].reshape(D, _TCOLS // 128, 128)
        for pp in range(_TCOLS // 1024):
            z = jnp.concatenate(
                [x3[:, pp * 8 + q, :] for q in range(8)], axis=0)
            out_ref[pl.ds(pp * 128, 128), :] = z.T


def _row_major4(W_cls, W_length, W_radian, W_loc):
    ospec = jax.ShapeDtypeStruct((_PROWS * D // 128, 128), jnp.float32)
    outs = pl.pallas_call(
        _transpose_block,
        grid=(_NBLK,),
        in_specs=[pl.BlockSpec((D, _TCOLS), lambda i: (0, i))] * 4,
        out_specs=[pl.BlockSpec((_TCOLS * D // 128, 128), lambda i: (i, 0))] * 4,
        out_shape=[ospec] * 4,
    )(W_cls.T, W_length.T, W_radian.T, W_loc.T)
    return [o.reshape(_PROWS, D) for o in outs]


def kernel(inputs, W_cls, W_length, W_radian, W_loc):
    idsT = inputs.astype(jnp.int32).T
    idsT = idsT[jnp.array([c for (_, c) in FIELDS])].reshape(-1)
    idsT = ((idsT & ~1023) | ((idsT & 127) << 3) | ((idsT >> 7) & 7))
    return _seg_embed(idsT, *_row_major4(W_cls, W_length, W_radian, W_loc))
